# baseline (speedup 1.0000x reference)
import dataclasses
import functools

import jax
import jax.numpy as jnp
from jax import lax
from jax.experimental import pallas as pl
from jax.experimental.pallas import tpu as pltpu
from jax.experimental.pallas import tpu_sc as plsc

MEMROWS = 100000
BATCH = 16384
D = 784
NC, NS, NLANE = 2, 16, 16
NW = NC * NS
CH = 2048
WIN = 160
NWIN = MEMROWS // WIN
BM = 1024
MB = 512

_sc_mesh = plsc.VectorSubcoreMesh(core_axis_name="c", subcore_axis_name="s")

_sc_params = pltpu.CompilerParams()
if "needs_layout_passes" in pltpu.CompilerParams.__dataclass_fields__:
    _sc_params = dataclasses.replace(_sc_params, needs_layout_passes=False)


@functools.partial(
    pl.kernel,
    out_type=(
        jax.ShapeDtypeStruct((MEMROWS, D), jnp.float32),
        jax.ShapeDtypeStruct((BATCH,), jnp.int32),
    ),
    mesh=_sc_mesh,
    compiler_params=_sc_params,
    scratch_types=[
        pltpu.VMEM((MEMROWS,), jnp.int32),
        pltpu.VMEM((CH,), jnp.int32),
        pltpu.VMEM((CH,), jnp.int32),
        pltpu.SemaphoreType.DMA,
    ],
)
def _k_sc(mem_hbm, idx_hbm, neg1_hbm, memcp_hbm, wtag_hbm, tab_v, idx_v,
          wtag_v, sem):
    w = lax.axis_index("c") * NS + lax.axis_index("s")

    @pl.when(w == 0)
    def _():
        pltpu.sync_copy(neg1_hbm, tab_v)
        lane = lax.iota(jnp.int32, NLANE)

        for p in range(3):
            @pl.loop(0, BATCH, step=CH)
            def _(c):
                pltpu.sync_copy(idx_hbm.at[pl.ds(c, CH)], idx_v)

                @pl.loop(0, CH, step=NLANE)
                def _(v):
                    iv = idx_v[pl.ds(v, NLANE)]
                    bv = (c + v) + lane
                    if p == 0:
                        plsc.store_scatter(tab_v, [iv], bv)
                    else:
                        cur = plsc.load_gather(tab_v, [iv])
                        plsc.store_scatter(tab_v, [iv], bv, mask=cur < bv)

        @pl.loop(0, BATCH, step=CH)
        def _(c):
            pltpu.sync_copy(idx_hbm.at[pl.ds(c, CH)], idx_v)

            @pl.loop(0, CH, step=NLANE)
            def _(v):
                iv = idx_v[pl.ds(v, NLANE)]
                bv = (c + v) + lane
                cur = plsc.load_gather(tab_v, [iv])
                wtag_v[pl.ds(v, NLANE)] = jnp.where(cur == bv, iv, -1)

            pltpu.sync_copy(wtag_v, wtag_hbm.at[pl.ds(c, CH)])

    @pl.when(w > 0)
    def _():
        @pl.loop(w - 1, NWIN, step=NW - 1)
        def _(t):
            pltpu.make_async_copy(
                mem_hbm.at[pl.ds(t * WIN, WIN)],
                memcp_hbm.at[pl.ds(t * WIN, WIN)],
                sem,
            ).start()

        @pl.loop(w - 1, NWIN, step=NW - 1)
        def _(t):
            pltpu.make_async_copy(
                mem_hbm.at[pl.ds(t * WIN, WIN)],
                memcp_hbm.at[pl.ds(t * WIN, WIN)],
                sem,
            ).wait()


def _k_mlp_body(idx_s, x_ref, mem_hbm, w1_ref, b1_ref, w2_ref, b2_ref,
                wrec_ref, brec_ref, wout_ref, bout_ref, out_ref, rows, sem):
    @pl.loop(0, BM)
    def _(j):
        pltpu.make_async_copy(
            mem_hbm.at[pl.ds(idx_s[j], 1)], rows.at[pl.ds(j, 1)], sem).start()

    hp = lax.Precision.HIGHEST
    act = jnp.maximum(
        jnp.dot(x_ref[...], w1_ref[...], precision=hp,
                preferred_element_type=jnp.float32) + b1_ref[...], 0.0)

    @pl.loop(0, BM)
    def _(j):
        pltpu.make_async_copy(
            mem_hbm.at[pl.ds(0, 1)], rows.at[pl.ds(j, 1)], sem).wait()

    a2 = (jnp.dot(act, w2_ref[...], precision=hp,
                  preferred_element_type=jnp.float32) + b2_ref[...]
          + jnp.dot(rows[...], wrec_ref[...], precision=hp,
                    preferred_element_type=jnp.float32) + brec_ref[...])
    a2 = jnp.maximum(a2, 0.0)
    out_ref[...] = jnp.dot(a2, wout_ref[...], precision=hp,
                           preferred_element_type=jnp.float32) + bout_ref[...]


def _k_mlp(idx, x, mem, w1, b1, w2, b2, wrec, brec, wout, bout):
    full = lambda a: pl.BlockSpec(a.shape, lambda i: (0,) * a.ndim)
    return pl.pallas_call(
        _k_mlp_body,
        grid=(BATCH // BM,),
        in_specs=[
            pl.BlockSpec((BM,), lambda i: (i,), memory_space=pltpu.SMEM),
            pl.BlockSpec((BM, D), lambda i: (i, 0)),
            pl.BlockSpec(memory_space=pl.ANY),
            full(w1), full(b1), full(w2), full(b2),
            full(wrec), full(brec), full(wout), full(bout),
        ],
        out_specs=pl.BlockSpec((BM, 10), lambda i: (i, 0)),
        out_shape=jax.ShapeDtypeStruct((BATCH, 10), jnp.float32),
        scratch_shapes=[pltpu.VMEM((BM, D), jnp.float32),
                        pltpu.SemaphoreType.DMA],
    )(idx, x, mem, w1, b1, w2, b2, wrec, brec, wout, bout)


def _k_merge_body(memcp_ref, wtag_s, x_hbm, out_ref, sem):
    step = pl.program_id(0)

    @pl.loop(0, MB)
    def _(j):
        t = wtag_s[j]

        @pl.when(t >= 0)
        def _():
            pltpu.make_async_copy(
                x_hbm.at[pl.ds(step * MB + j, 1)],
                out_ref.at[pl.ds(t, 1)],
                sem,
            ).start()

    @pl.loop(0, MB)
    def _(j):
        @pl.when(wtag_s[j] >= 0)
        def _():
            pltpu.make_async_copy(
                x_hbm.at[pl.ds(0, 1)], out_ref.at[pl.ds(0, 1)], sem
            ).wait()


def _k_merge(mem_cp, wtag, x):
    return pl.pallas_call(
        _k_merge_body,
        grid=(BATCH // MB,),
        in_specs=[
            pl.BlockSpec(memory_space=pl.ANY),
            pl.BlockSpec((MB,), lambda i: (i,), memory_space=pltpu.SMEM),
            pl.BlockSpec(memory_space=pl.ANY),
        ],
        out_specs=pl.BlockSpec(memory_space=pl.ANY),
        out_shape=jax.ShapeDtypeStruct((MEMROWS, D), jnp.float32),
        scratch_shapes=[pltpu.SemaphoreType.DMA],
        input_output_aliases={0: 0},
        compiler_params=pltpu.CompilerParams(has_side_effects=True),
    )(mem_cp, wtag, x)


def kernel(x_sensory, mem_vals, W1, b1, W2, b2, Wrec, brec, Wout, bout):
    h = lax.stop_gradient(jax.nn.relu(x_sensory @ W1 + b1))
    mult = jnp.arange(1, h.shape[1] + 1, dtype=jnp.float32) * 2654435.0
    code = jnp.floor(h * 8.0) @ mult
    idx = jnp.mod(jnp.abs(code), float(MEMROWS))
    idx = jnp.clip(idx.astype(jnp.int32), 0, MEMROWS - 1)

    neg1 = jnp.full((MEMROWS,), -1, jnp.int32)
    mem_cp, wtag = _k_sc(mem_vals, idx, neg1)

    out = _k_mlp(idx, x_sensory, mem_vals, W1, b1.reshape(1, -1), W2,
                 b2.reshape(1, -1), Wrec, brec.reshape(1, -1), Wout,
                 bout.reshape(1, -1))

    mem_new = _k_merge(mem_cp, wtag, x_sensory)
    return out, mem_new


# speedup vs baseline: 2.4127x; 2.4127x over previous
import dataclasses
import functools

import jax
import jax.numpy as jnp
from jax import lax
from jax.experimental import pallas as pl
from jax.experimental.pallas import tpu as pltpu
from jax.experimental.pallas import tpu_sc as plsc

MEMROWS = 100000
MEMG = MEMROWS // 8
BATCH = 16384
BATG = BATCH // 8
D = 784
NC, NS, NLANE = 2, 16, 16
NW = NC * NS
BPW = BATCH // NW
CH = 2048
TSH = 3136
TABN = NW * TSH
WING = 10
NWIN = MEMG // WING
WROWS = WING * 8
BM = 1024

_sc_mesh = plsc.VectorSubcoreMesh(core_axis_name="c", subcore_axis_name="s")

_sc_params = pltpu.CompilerParams()
if "needs_layout_passes" in pltpu.CompilerParams.__dataclass_fields__:
    _sc_params = dataclasses.replace(_sc_params, needs_layout_passes=False)


def _start_row_dma(src3, g, s, dst3, dg, ds_, sem):
    dyn_s = not isinstance(s, int)
    dyn_d = not isinstance(ds_, int)
    for sv in range(8):
        if not dyn_s and sv != s:
            continue
        for dv in range(8):
            if not dyn_d and dv != ds_:
                continue

            def start():
                pltpu.make_async_copy(
                    src3.at[pl.ds(g, 1), pl.ds(sv, 1)],
                    dst3.at[pl.ds(dg, 1), pl.ds(dv, 1)],
                    sem,
                ).start()

            if dyn_s and dyn_d:
                pl.when((s == sv) & (ds_ == dv))(start)
            elif dyn_s:
                pl.when(s == sv)(start)
            elif dyn_d:
                pl.when(ds_ == dv)(start)
            else:
                start()


@functools.partial(
    pl.kernel,
    out_type=jax.ShapeDtypeStruct((TABN,), jnp.int32),
    mesh=_sc_mesh,
    compiler_params=_sc_params,
    scratch_types=[
        pltpu.VMEM((TSH,), jnp.int32),
        pltpu.VMEM((CH,), jnp.int32),
    ],
)
def _k_tab(idx_hbm, tab_hbm, tab_v, idx_v):
    w = lax.axis_index("c") * NS + lax.axis_index("s")
    lo = w * TSH
    hi = lo + TSH
    lane = lax.iota(jnp.int32, NLANE)

    @pl.loop(0, TSH, step=NLANE)
    def _(v):
        tab_v[pl.ds(v, NLANE)] = jnp.full((NLANE,), -1, jnp.int32)

    for p in range(3):
        @pl.loop(0, BATCH, step=CH)
        def _(c):
            pltpu.sync_copy(idx_hbm.at[pl.ds(c, CH)], idx_v)

            @pl.loop(0, CH, step=NLANE)
            def _(v):
                iv = idx_v[pl.ds(v, NLANE)]
                bv = (c + v) + lane
                mine = (iv >= lo) & (iv < hi)
                ivl = jnp.clip(iv - lo, 0, TSH - 1)
                if p == 0:
                    plsc.store_scatter(tab_v, [ivl], bv, mask=mine)
                else:
                    cur = plsc.load_gather(tab_v, [ivl], mask=mine)
                    plsc.store_scatter(tab_v, [ivl], bv,
                                       mask=mine & (cur < bv))

    pltpu.sync_copy(tab_v, tab_hbm.at[pl.ds(lo, TSH)])


@functools.partial(
    pl.kernel,
    out_type=(
        jax.ShapeDtypeStruct((MEMG, 8, D), jnp.float32),
        jax.ShapeDtypeStruct((BATG, 8, D), jnp.float32),
    ),
    mesh=_sc_mesh,
    compiler_params=_sc_params,
    scratch_types=[
        pltpu.VMEM((WING, 8, D), jnp.float32),
        pltpu.VMEM((BPW,), jnp.int32),
        pltpu.VMEM((WROWS,), jnp.int32),
        pltpu.SemaphoreType.DMA,
        pltpu.SemaphoreType.DMA,
    ],
)
def _k_gm(mem3_hbm, x3_hbm, idx_hbm, tab_hbm, memnew_hbm, xrec_hbm,
          buf_v, idx_v, tabw_v, sem_w, sem_r):
    w = lax.axis_index("c") * NS + lax.axis_index("s")

    base = w * BPW
    pltpu.sync_copy(idx_hbm.at[pl.ds(base, BPW)], idx_v)

    @pl.loop(0, BPW, step=NLANE)
    def _(j0):
        vals = idx_v[pl.ds(j0, NLANE)]
        for k in range(NLANE):
            slot = vals[k]
            _start_row_dma(mem3_hbm, slot // 8, slot % 8,
                           xrec_hbm, (base + j0 + k) // 8, (j0 + k) % 8,
                           sem_r)

        for k in range(NLANE):
            pltpu.make_async_copy(
                mem3_hbm.at[pl.ds(0, 1), pl.ds(0, 1)],
                xrec_hbm.at[pl.ds((base + j0 + k) // 8, 1),
                            pl.ds((j0 + k) % 8, 1)],
                sem_r,
            ).wait()

    @pl.loop(w, NWIN, step=NW)
    def _(t):
        g0 = t * WING
        cp_in = pltpu.make_async_copy(
            mem3_hbm.at[pl.ds(g0, WING)], buf_v, sem_w)
        cp_in.start()
        pltpu.sync_copy(tab_hbm.at[pl.ds(g0 * 8, WROWS)], tabw_v)
        cp_in.wait()

        for k0 in range(0, WROWS, NLANE):
            tags = tabw_v[pl.ds(k0, NLANE)]
            for k in range(NLANE):
                src = tags[k]
                jg, js = (k0 + k) // 8, (k0 + k) % 8

                @pl.when(src >= 0)
                def _():
                    _start_row_dma(x3_hbm, src // 8, src % 8,
                                   buf_v, jg, js, sem_r)

        for k0 in range(0, WROWS, NLANE):
            tags = tabw_v[pl.ds(k0, NLANE)]
            for k in range(NLANE):
                @pl.when(tags[k] >= 0)
                def _():
                    pltpu.make_async_copy(
                        x3_hbm.at[pl.ds(0, 1), pl.ds(0, 1)],
                        buf_v.at[pl.ds(0, 1), pl.ds(0, 1)],
                        sem_r,
                    ).wait()

        pltpu.sync_copy(buf_v, memnew_hbm.at[pl.ds(g0, WING)])


def _k_mlp_body(x_ref, xrec_ref, w1_ref, b1_ref, w2_ref, b2_ref, wrec_ref,
                brec_ref, wout_ref, bout_ref, out_ref):
    hp = lax.Precision.HIGHEST
    act = jnp.maximum(
        jnp.dot(x_ref[...], w1_ref[...], precision=hp,
                preferred_element_type=jnp.float32) + b1_ref[...], 0.0)
    a2 = (jnp.dot(act, w2_ref[...], precision=hp,
                  preferred_element_type=jnp.float32) + b2_ref[...]
          + jnp.dot(xrec_ref[...], wrec_ref[...], precision=hp,
                    preferred_element_type=jnp.float32) + brec_ref[...])
    a2 = jnp.maximum(a2, 0.0)
    out_ref[...] = jnp.dot(a2, wout_ref[...], precision=hp,
                           preferred_element_type=jnp.float32) + bout_ref[...]


def _k_mlp(x, xrec, w1, b1, w2, b2, wrec, brec, wout, bout):
    full = lambda a: pl.BlockSpec(a.shape, lambda i: (0,) * a.ndim)
    return pl.pallas_call(
        _k_mlp_body,
        grid=(BATCH // BM,),
        in_specs=[
            pl.BlockSpec((BM, D), lambda i: (i, 0)),
            pl.BlockSpec((BM, D), lambda i: (i, 0)),
            full(w1), full(b1), full(w2), full(b2),
            full(wrec), full(brec), full(wout), full(bout),
        ],
        out_specs=pl.BlockSpec((BM, 10), lambda i: (i, 0)),
        out_shape=jax.ShapeDtypeStruct((BATCH, 10), jnp.float32),
    )(x, xrec, w1, b1, w2, b2, wrec, brec, wout, bout)


def kernel(x_sensory, mem_vals, W1, b1, W2, b2, Wrec, brec, Wout, bout):
    h = lax.stop_gradient(jax.nn.relu(x_sensory @ W1 + b1))
    mult = jnp.arange(1, h.shape[1] + 1, dtype=jnp.float32) * 2654435.0
    code = jnp.floor(h * 8.0) @ mult
    idx = jnp.mod(jnp.abs(code), float(MEMROWS))
    idx = jnp.clip(idx.astype(jnp.int32), 0, MEMROWS - 1)

    mem3 = mem_vals.reshape(MEMG, 8, D)
    x3 = x_sensory.reshape(BATG, 8, D)

    tab = _k_tab(idx)
    mem_new3, xrec3 = _k_gm(mem3, x3, idx, tab)

    out = _k_mlp(x_sensory, xrec3.reshape(BATCH, D), W1, b1.reshape(1, -1),
                 W2, b2.reshape(1, -1), Wrec, brec.reshape(1, -1), Wout,
                 bout.reshape(1, -1))

    return out, mem_new3.reshape(MEMROWS, D)


# speedup vs baseline: 2.7923x; 1.1573x over previous
import dataclasses
import functools

import jax
import jax.numpy as jnp
from jax import lax
from jax.experimental import pallas as pl
from jax.experimental.pallas import tpu as pltpu
from jax.experimental.pallas import tpu_sc as plsc

MEMROWS = 100000
MEMG = MEMROWS // 8
BATCH = 16384
BATG = BATCH // 8
D = 784
NC, NS, NLANE = 2, 16, 16
NW = NC * NS
BPW = BATCH // NW
CH = 2048
TSH = 3136
TABN = NW * TSH
WING = 10
NWIN = MEMG // WING
WROWS = WING * 8
BM = 1024

_sc_mesh = plsc.VectorSubcoreMesh(core_axis_name="c", subcore_axis_name="s")

_sc_params = pltpu.CompilerParams()
if "needs_layout_passes" in pltpu.CompilerParams.__dataclass_fields__:
    _sc_params = dataclasses.replace(_sc_params, needs_layout_passes=False)


def _start_row_dma(src3, g, s, dst3, dg, ds_, sem):
    dyn_s = not isinstance(s, int)
    dyn_d = not isinstance(ds_, int)
    for sv in range(8):
        if not dyn_s and sv != s:
            continue
        for dv in range(8):
            if not dyn_d and dv != ds_:
                continue

            def start():
                pltpu.make_async_copy(
                    src3.at[pl.ds(g, 1), pl.ds(sv, 1)],
                    dst3.at[pl.ds(dg, 1), pl.ds(dv, 1)],
                    sem,
                ).start()

            if dyn_s and dyn_d:
                pl.when((s == sv) & (ds_ == dv))(start)
            elif dyn_s:
                pl.when(s == sv)(start)
            elif dyn_d:
                pl.when(ds_ == dv)(start)
            else:
                start()


@functools.partial(
    pl.kernel,
    out_type=jax.ShapeDtypeStruct((TABN,), jnp.int32),
    mesh=_sc_mesh,
    compiler_params=_sc_params,
    scratch_types=[
        pltpu.VMEM((TSH,), jnp.int32),
        pltpu.VMEM((CH,), jnp.int32),
    ],
)
def _k_tab(idx_hbm, tab_hbm, tab_v, idx_v):
    w = lax.axis_index("c") * NS + lax.axis_index("s")
    lo = w * TSH
    hi = lo + TSH
    lane = lax.iota(jnp.int32, NLANE)

    @pl.loop(0, TSH, step=NLANE)
    def _(v):
        tab_v[pl.ds(v, NLANE)] = jnp.full((NLANE,), -1, jnp.int32)

    for p in range(3):
        @pl.loop(0, BATCH, step=CH)
        def _(c):
            pltpu.sync_copy(idx_hbm.at[pl.ds(c, CH)], idx_v)

            @pl.loop(0, CH, step=NLANE)
            def _(v):
                iv = idx_v[pl.ds(v, NLANE)]
                bv = (c + v) + lane
                mine = (iv >= lo) & (iv < hi)
                ivl = jnp.clip(iv - lo, 0, TSH - 1)
                if p == 0:
                    plsc.store_scatter(tab_v, [ivl], bv, mask=mine)
                else:
                    cur = plsc.load_gather(tab_v, [ivl], mask=mine)
                    plsc.store_scatter(tab_v, [ivl], bv,
                                       mask=mine & (cur < bv))

    pltpu.sync_copy(tab_v, tab_hbm.at[pl.ds(lo, TSH)])


@functools.partial(
    pl.kernel,
    out_type=jax.ShapeDtypeStruct((BATG, 8, D), jnp.float32),
    mesh=_sc_mesh,
    compiler_params=_sc_params,
    scratch_types=[
        pltpu.VMEM((BPW,), jnp.int32),
        pltpu.SemaphoreType.DMA,
    ],
)
def _k_ga(mem3_hbm, idx_hbm, xrec_hbm, idx_v, sem_r):
    w = lax.axis_index("c") * NS + lax.axis_index("s")

    base = w * BPW
    pltpu.sync_copy(idx_hbm.at[pl.ds(base, BPW)], idx_v)

    @pl.loop(0, BPW, step=NLANE)
    def _(j0):
        vals = idx_v[pl.ds(j0, NLANE)]
        for k in range(NLANE):
            slot = vals[k]
            _start_row_dma(mem3_hbm, slot // 8, slot % 8,
                           xrec_hbm, (base + j0 + k) // 8, (j0 + k) % 8,
                           sem_r)

        for k in range(NLANE):
            pltpu.make_async_copy(
                mem3_hbm.at[pl.ds(0, 1), pl.ds(0, 1)],
                xrec_hbm.at[pl.ds((base + j0 + k) // 8, 1),
                            pl.ds((j0 + k) % 8, 1)],
                sem_r,
            ).wait()


TB = 512
NTB = 196


def _trans_body(in_ref, out_ref):
    out_ref[...] = in_ref[...].T


def _transpose_to_rm(at, nrows):
    nblk = (nrows + TB - 1) // TB
    return pl.pallas_call(
        _trans_body,
        grid=(nblk,),
        in_specs=[pl.BlockSpec((D, TB), lambda i: (0, i))],
        out_specs=pl.BlockSpec((TB, D), lambda i: (i, 0)),
        out_shape=jax.ShapeDtypeStruct((nrows, D), jnp.float32),
    )(at)


def _mgtr_body(tab_s, memrm_ref, x_hbm, out_ref, blk, sem):
    blk[...] = memrm_ref[...]

    @pl.loop(0, TB)
    def _(j):
        t = tab_s[j]

        @pl.when(t >= 0)
        def _():
            pltpu.make_async_copy(
                x_hbm.at[pl.ds(t, 1)], blk.at[pl.ds(j, 1)], sem).start()

    @pl.loop(0, TB)
    def _(j):
        @pl.when(tab_s[j] >= 0)
        def _():
            pltpu.make_async_copy(
                x_hbm.at[pl.ds(0, 1)], blk.at[pl.ds(0, 1)], sem).wait()

    out_ref[...] = blk[...].T


def _k_mgtr(tab, mem_rm, x_rm):
    return pl.pallas_call(
        _mgtr_body,
        grid=(NTB,),
        in_specs=[
            pl.BlockSpec((TB,), lambda i: (i,), memory_space=pltpu.SMEM),
            pl.BlockSpec((TB, D), lambda i: (i, 0)),
            pl.BlockSpec(memory_space=pl.ANY),
        ],
        out_specs=pl.BlockSpec((D, TB), lambda i: (0, i)),
        out_shape=jax.ShapeDtypeStruct((D, MEMROWS), jnp.float32),
        scratch_shapes=[pltpu.VMEM((TB, D), jnp.float32),
                        pltpu.SemaphoreType.DMA],
    )(tab, mem_rm, x_rm)


def _k_mlp_body(x_ref, xrec_ref, w1_ref, b1_ref, w2_ref, b2_ref, wrec_ref,
                brec_ref, wout_ref, bout_ref, out_ref):
    hp = lax.Precision.HIGHEST
    act = jnp.maximum(
        jnp.dot(x_ref[...], w1_ref[...], precision=hp,
                preferred_element_type=jnp.float32) + b1_ref[...], 0.0)
    a2 = (jnp.dot(act, w2_ref[...], precision=hp,
                  preferred_element_type=jnp.float32) + b2_ref[...]
          + jnp.dot(xrec_ref[...], wrec_ref[...], precision=hp,
                    preferred_element_type=jnp.float32) + brec_ref[...])
    a2 = jnp.maximum(a2, 0.0)
    out_ref[...] = jnp.dot(a2, wout_ref[...], precision=hp,
                           preferred_element_type=jnp.float32) + bout_ref[...]


def _k_mlp(x, xrec, w1, b1, w2, b2, wrec, brec, wout, bout):
    full = lambda a: pl.BlockSpec(a.shape, lambda i: (0,) * a.ndim)
    return pl.pallas_call(
        _k_mlp_body,
        grid=(BATCH // BM,),
        in_specs=[
            pl.BlockSpec((BM, D), lambda i: (i, 0)),
            pl.BlockSpec((BM, D), lambda i: (i, 0)),
            full(w1), full(b1), full(w2), full(b2),
            full(wrec), full(brec), full(wout), full(bout),
        ],
        out_specs=pl.BlockSpec((BM, 10), lambda i: (i, 0)),
        out_shape=jax.ShapeDtypeStruct((BATCH, 10), jnp.float32),
    )(x, xrec, w1, b1, w2, b2, wrec, brec, wout, bout)


def kernel(x_sensory, mem_vals, W1, b1, W2, b2, Wrec, brec, Wout, bout):
    h = lax.stop_gradient(jax.nn.relu(x_sensory @ W1 + b1))
    mult = jnp.arange(1, h.shape[1] + 1, dtype=jnp.float32) * 2654435.0
    code = jnp.floor(h * 8.0) @ mult
    idx = jnp.mod(jnp.abs(code), float(MEMROWS))
    idx = jnp.clip(idx.astype(jnp.int32), 0, MEMROWS - 1)

    mem_rm = _transpose_to_rm(mem_vals.T, MEMROWS)
    x_rm = _transpose_to_rm(x_sensory.T, BATCH)

    tab = _k_tab(idx)
    xrec3 = _k_ga(mem_rm.reshape(MEMG, 8, D), idx)

    out = _k_mlp(x_rm, xrec3.reshape(BATCH, D), W1, b1.reshape(1, -1),
                 W2, b2.reshape(1, -1), Wrec, brec.reshape(1, -1), Wout,
                 bout.reshape(1, -1))

    mem_newT = _k_mgtr(tab, mem_rm, x_rm)
    return out, mem_newT.T


# speedup vs baseline: 4.4546x; 1.5953x over previous
import dataclasses
import functools

import jax
import jax.numpy as jnp
from jax import lax
from jax.experimental import pallas as pl
from jax.experimental.pallas import tpu as pltpu
from jax.experimental.pallas import tpu_sc as plsc

MEMROWS = 100000
MEMG = MEMROWS // 8
BATCH = 16384
BATG = BATCH // 8
D = 784
NC, NS, NLANE = 2, 16, 16
NW = NC * NS
BPW = BATCH // NW
CH = 2048
TSH = 3136
TABN = NW * TSH
WING = 10
NWIN = MEMG // WING
WROWS = WING * 8
BM = 1024
DP = 896

_sc_mesh = plsc.VectorSubcoreMesh(core_axis_name="c", subcore_axis_name="s")

_sc_params = pltpu.CompilerParams()
if "needs_layout_passes" in pltpu.CompilerParams.__dataclass_fields__:
    _sc_params = dataclasses.replace(_sc_params, needs_layout_passes=False)


def _start_row_dma(src3, g, s, dst3, dg, ds_, sem):
    dyn_s = not isinstance(s, int)
    dyn_d = not isinstance(ds_, int)
    for sv in range(8):
        if not dyn_s and sv != s:
            continue
        for dv in range(8):
            if not dyn_d and dv != ds_:
                continue

            def start():
                pltpu.make_async_copy(
                    src3.at[pl.ds(g, 1), pl.ds(sv, 1)],
                    dst3.at[pl.ds(dg, 1), pl.ds(dv, 1)],
                    sem,
                ).start()

            if dyn_s and dyn_d:
                pl.when((s == sv) & (ds_ == dv))(start)
            elif dyn_s:
                pl.when(s == sv)(start)
            elif dyn_d:
                pl.when(ds_ == dv)(start)
            else:
                start()


@functools.partial(
    pl.kernel,
    out_type=jax.ShapeDtypeStruct((TABN,), jnp.int32),
    mesh=_sc_mesh,
    compiler_params=_sc_params,
    scratch_types=[
        pltpu.VMEM((TSH,), jnp.int32),
        pltpu.VMEM((CH,), jnp.int32),
    ],
)
def _k_tab(idx_hbm, tab_hbm, tab_v, idx_v):
    w = lax.axis_index("c") * NS + lax.axis_index("s")
    lo = w * TSH
    hi = lo + TSH
    lane = lax.iota(jnp.int32, NLANE)

    @pl.loop(0, TSH, step=NLANE)
    def _(v):
        tab_v[pl.ds(v, NLANE)] = jnp.full((NLANE,), -1, jnp.int32)

    for p in range(3):
        @pl.loop(0, BATCH, step=CH)
        def _(c):
            pltpu.sync_copy(idx_hbm.at[pl.ds(c, CH)], idx_v)

            @pl.loop(0, CH, step=NLANE)
            def _(v):
                iv = idx_v[pl.ds(v, NLANE)]
                bv = (c + v) + lane
                mine = (iv >= lo) & (iv < hi)
                ivl = jnp.clip(iv - lo, 0, TSH - 1)
                if p == 0:
                    plsc.store_scatter(tab_v, [ivl], bv, mask=mine)
                else:
                    cur = plsc.load_gather(tab_v, [ivl], mask=mine)
                    plsc.store_scatter(tab_v, [ivl], bv,
                                       mask=mine & (cur < bv))

    pltpu.sync_copy(tab_v, tab_hbm.at[pl.ds(lo, TSH)])


@functools.partial(
    pl.kernel,
    out_type=jax.ShapeDtypeStruct((BATG, 8, DP), jnp.float32),
    mesh=_sc_mesh,
    compiler_params=_sc_params,
    scratch_types=[
        pltpu.VMEM((BPW,), jnp.int32),
        pltpu.VMEM((BPW,), jnp.int32),
        pltpu.VMEM((8, 8, DP), jnp.float32),
        pltpu.VMEM_SHARED((NS, 2, 8, DP), jnp.float32),
        pltpu.SemaphoreType.DMA,
    ],
)
def _k_ga(mem3_hbm, idx_hbm, xrec_hbm, idx_v, gidx_v, grp_v, rows_s, sem_r):
    sid = lax.axis_index("s")
    w = lax.axis_index("c") * NS + lax.axis_index("s")

    base = w * BPW
    pltpu.sync_copy(idx_hbm.at[pl.ds(base, BPW)], idx_v)

    @pl.loop(0, BPW, step=NLANE)
    def _(j0):
        gidx_v[pl.ds(j0, NLANE)] = idx_v[pl.ds(j0, NLANE)] // 8

    @pl.loop(0, BPW, step=NLANE)
    def _(j0):
        vals = idx_v[pl.ds(j0, NLANE)]
        for h in range(2):
            pltpu.sync_copy(mem3_hbm.at[gidx_v.at[pl.ds(j0 + 8 * h, 8)]],
                            grp_v)
            for k in range(8):
                slot = vals[8 * h + k]
                _start_row_dma(grp_v, k, slot % 8,
                               rows_s.at[sid], h, k, sem_r)

            for k in range(8):
                pltpu.make_async_copy(
                    grp_v.at[pl.ds(0, 1), pl.ds(0, 1)],
                    rows_s.at[pl.ds(sid, 1), pl.ds(h, 1), pl.ds(k, 1)],
                    sem_r,
                ).wait()

        pltpu.sync_copy(rows_s.at[sid], xrec_hbm.at[pl.ds((base + j0) // 8, 2)])


TB = 512
NTB = 196


def _trans_body(in_ref, out_ref):
    out_ref[...] = in_ref[...].T


def _transpose_to_rm(at, nrows):
    nblk = (nrows + TB - 1) // TB
    return pl.pallas_call(
        _trans_body,
        grid=(nblk,),
        in_specs=[pl.BlockSpec((D, TB), lambda i: (0, i))],
        out_specs=pl.BlockSpec((TB, D), lambda i: (i, 0)),
        out_shape=jax.ShapeDtypeStruct((nrows, D), jnp.float32),
    )(at)


def _trans_pad_body(in_ref, out_ref):
    t = in_ref[...].T
    z = jnp.zeros((TB, DP - D), jnp.float32)
    out_ref[...] = jnp.concatenate([t, z], axis=1).reshape(TB // 8, 8, DP)


def _transpose_mem_rm(at):
    return pl.pallas_call(
        _trans_pad_body,
        grid=(NTB,),
        in_specs=[pl.BlockSpec((D, TB), lambda i: (0, i))],
        out_specs=pl.BlockSpec((TB // 8, 8, DP), lambda i: (i, 0, 0)),
        out_shape=jax.ShapeDtypeStruct((MEMG, 8, DP), jnp.float32),
    )(at)


def _mgtr_body(tab_s, memrm_ref, x_hbm, out_ref, blk, sem):
    blk[...] = memrm_ref[...].reshape(TB, DP)[:, :D]

    @pl.loop(0, TB)
    def _(j):
        t = tab_s[j]

        @pl.when(t >= 0)
        def _():
            pltpu.make_async_copy(
                x_hbm.at[pl.ds(t, 1)], blk.at[pl.ds(j, 1)], sem).start()

    @pl.loop(0, TB)
    def _(j):
        @pl.when(tab_s[j] >= 0)
        def _():
            pltpu.make_async_copy(
                x_hbm.at[pl.ds(0, 1)], blk.at[pl.ds(0, 1)], sem).wait()

    out_ref[...] = blk[...].T


def _k_mgtr(tab, mem_rm, x_rm):
    return pl.pallas_call(
        _mgtr_body,
        grid=(NTB,),
        in_specs=[
            pl.BlockSpec((TB,), lambda i: (i,), memory_space=pltpu.SMEM),
            pl.BlockSpec((TB // 8, 8, DP), lambda i: (i, 0, 0)),
            pl.BlockSpec(memory_space=pl.ANY),
        ],
        out_specs=pl.BlockSpec((D, TB), lambda i: (0, i)),
        out_shape=jax.ShapeDtypeStruct((D, MEMROWS), jnp.float32),
        scratch_shapes=[pltpu.VMEM((TB, D), jnp.float32),
                        pltpu.SemaphoreType.DMA],
    )(tab, mem_rm, x_rm)


def _k_mlp_body(x_ref, xrec_ref, w1_ref, b1_ref, w2_ref, b2_ref, wrec_ref,
                brec_ref, wout_ref, bout_ref, out_ref):
    act = jnp.maximum(
        jnp.dot(x_ref[...], w1_ref[...],
                preferred_element_type=jnp.float32) + b1_ref[...], 0.0)
    a2 = (jnp.dot(act, w2_ref[...],
                  preferred_element_type=jnp.float32) + b2_ref[...]
          + jnp.dot(xrec_ref[...], wrec_ref[...],
                    preferred_element_type=jnp.float32) + brec_ref[...])
    a2 = jnp.maximum(a2, 0.0)
    out_ref[...] = jnp.dot(a2, wout_ref[...],
                           preferred_element_type=jnp.float32) + bout_ref[...]


def _k_mlp(x, xrec, w1, b1, w2, b2, wrec, brec, wout, bout):
    full = lambda a: pl.BlockSpec(a.shape, lambda i: (0,) * a.ndim)
    return pl.pallas_call(
        _k_mlp_body,
        grid=(BATCH // BM,),
        in_specs=[
            pl.BlockSpec((BM, D), lambda i: (i, 0)),
            pl.BlockSpec((BM, DP), lambda i: (i, 0)),
            full(w1), full(b1), full(w2), full(b2),
            full(wrec), full(brec), full(wout), full(bout),
        ],
        out_specs=pl.BlockSpec((BM, 10), lambda i: (i, 0)),
        out_shape=jax.ShapeDtypeStruct((BATCH, 10), jnp.float32),
    )(x, xrec, w1, b1, w2, b2, wrec, brec, wout, bout)


def kernel(x_sensory, mem_vals, W1, b1, W2, b2, Wrec, brec, Wout, bout):
    h = lax.stop_gradient(jax.nn.relu(x_sensory @ W1 + b1))
    mult = jnp.arange(1, h.shape[1] + 1, dtype=jnp.float32) * 2654435.0
    code = jnp.floor(h * 8.0) @ mult
    idx = jnp.mod(jnp.abs(code), float(MEMROWS))
    idx = jnp.clip(idx.astype(jnp.int32), 0, MEMROWS - 1)

    mem_rm = _transpose_mem_rm(mem_vals.T)
    x_rm = _transpose_to_rm(x_sensory.T, BATCH)

    tab = _k_tab(idx)
    xrec3 = _k_ga(mem_rm, idx)

    wrec_pad = jnp.pad(Wrec, ((0, DP - D), (0, 0)))
    out = _k_mlp(x_rm, xrec3.reshape(BATCH, DP), W1, b1.reshape(1, -1),
                 W2, b2.reshape(1, -1), wrec_pad, brec.reshape(1, -1), Wout,
                 bout.reshape(1, -1))

    mem_newT = _k_mgtr(tab, mem_rm, x_rm)
    return out, mem_newT.T


# speedup vs baseline: 9.4063x; 2.1116x over previous
import dataclasses
import functools

import jax
import jax.numpy as jnp
from jax import lax
from jax.experimental import pallas as pl
from jax.experimental.pallas import tpu as pltpu
from jax.experimental.pallas import tpu_sc as plsc

MEMROWS = 100000
MEMG = MEMROWS // 8
BATCH = 16384
BATG = BATCH // 8
D = 784
NC, NS, NLANE = 2, 16, 16
NW = NC * NS
BPW = BATCH // NW
CH = 2048
TSH = 3584
TABN = NW * TSH
WING = 10
NWIN = MEMG // WING
WROWS = WING * 8
BM = 1024
DP = 896
TBM = 512
NBM = TABN // TBM
WREG = 256

_sc_mesh = plsc.VectorSubcoreMesh(core_axis_name="c", subcore_axis_name="s")

_sc_params = pltpu.CompilerParams()
if "needs_layout_passes" in pltpu.CompilerParams.__dataclass_fields__:
    _sc_params = dataclasses.replace(_sc_params, needs_layout_passes=False)


def _start_row_dma(src3, g, s, dst3, dg, ds_, sem):
    dyn_s = not isinstance(s, int)
    dyn_d = not isinstance(ds_, int)
    for sv in range(8):
        if not dyn_s and sv != s:
            continue
        for dv in range(8):
            if not dyn_d and dv != ds_:
                continue

            def start():
                pltpu.make_async_copy(
                    src3.at[pl.ds(g, 1), pl.ds(sv, 1)],
                    dst3.at[pl.ds(dg, 1), pl.ds(dv, 1)],
                    sem,
                ).start()

            if dyn_s and dyn_d:
                pl.when((s == sv) & (ds_ == dv))(start)
            elif dyn_s:
                pl.when(s == sv)(start)
            elif dyn_d:
                pl.when(ds_ == dv)(start)
            else:
                start()


@functools.partial(
    pl.kernel,
    out_type=(
        jax.ShapeDtypeStruct((NBM * WREG,), jnp.int32),
        jax.ShapeDtypeStruct((NBM * WREG,), jnp.int32),
        jax.ShapeDtypeStruct((NW * NLANE,), jnp.int32),
    ),
    mesh=_sc_mesh,
    compiler_params=_sc_params,
    scratch_types=[
        pltpu.VMEM((TSH,), jnp.int32),
        pltpu.VMEM((CH,), jnp.int32),
        pltpu.VMEM((WREG,), jnp.int32),
        pltpu.VMEM((WREG,), jnp.int32),
        pltpu.VMEM((NLANE,), jnp.int32),
    ],
)
def _k_tab(idx_hbm, wsl_hbm, wsr_hbm, wcnt_hbm, tab_v, idx_v, wsl_v, wsr_v,
           cnt_v):
    w = lax.axis_index("c") * NS + lax.axis_index("s")
    lo = w * TSH
    hi = lo + TSH
    lane = lax.iota(jnp.int32, NLANE)

    @pl.loop(0, TSH, step=NLANE)
    def _(v):
        tab_v[pl.ds(v, NLANE)] = jnp.full((NLANE,), -1, jnp.int32)

    for p in range(3):
        @pl.loop(0, BATCH, step=CH)
        def _(c):
            pltpu.sync_copy(idx_hbm.at[pl.ds(c, CH)], idx_v)

            @pl.loop(0, CH, step=NLANE)
            def _(v):
                iv = idx_v[pl.ds(v, NLANE)]
                bv = (c + v) + lane
                mine = (iv >= lo) & (iv < hi)
                ivl = jnp.clip(iv - lo, 0, TSH - 1)
                if p == 0:
                    plsc.store_scatter(tab_v, [ivl], bv, mask=mine)
                else:
                    cur = plsc.load_gather(tab_v, [ivl], mask=mine)
                    plsc.store_scatter(tab_v, [ivl], bv,
                                       mask=mine & (cur < bv))

    cv = jnp.zeros((NLANE,), jnp.int32)
    for r in range(7):
        base_local = r * TBM

        def scan_body(v49, cnt):
            off = base_local + v49 * NLANE
            tv = tab_v[pl.ds(off, NLANE)]
            m = tv >= 0
            cm = plsc.cumsum(jnp.where(m, 1, 0).astype(jnp.int32))
            pos = cnt + cm - 1
            slotv = (off - base_local) + lane
            plsc.store_scatter(wsl_v, [pos], slotv, mask=m)
            plsc.store_scatter(wsr_v, [pos], tv, mask=m)
            return cnt + plsc.all_reduce_population_count(m)[0]

        cnt = lax.fori_loop(0, TBM // NLANE, scan_body, jnp.int32(0))
        b = w * 7 + r
        pltpu.sync_copy(wsl_v, wsl_hbm.at[pl.ds(b * WREG, WREG)])
        pltpu.sync_copy(wsr_v, wsr_hbm.at[pl.ds(b * WREG, WREG)])
        cv = jnp.where(lane == r, cnt, cv)

    cnt_v[...] = cv
    pltpu.sync_copy(cnt_v, wcnt_hbm.at[pl.ds(w * NLANE, NLANE)])


@functools.partial(
    pl.kernel,
    out_type=jax.ShapeDtypeStruct((BATG, 8, DP), jnp.float32),
    mesh=_sc_mesh,
    compiler_params=_sc_params,
    scratch_types=[
        pltpu.VMEM((BPW,), jnp.int32),
        pltpu.VMEM((BPW,), jnp.int32),
        pltpu.VMEM((8, 8, DP), jnp.float32),
        pltpu.VMEM_SHARED((NS, 2, 8, DP), jnp.float32),
        pltpu.SemaphoreType.DMA,
    ],
)
def _k_ga(mem3_hbm, idx_hbm, xrec_hbm, idx_v, gidx_v, grp_v, rows_s, sem_r):
    sid = lax.axis_index("s")
    w = lax.axis_index("c") * NS + lax.axis_index("s")

    base = w * BPW
    pltpu.sync_copy(idx_hbm.at[pl.ds(base, BPW)], idx_v)

    @pl.loop(0, BPW, step=NLANE)
    def _(j0):
        gidx_v[pl.ds(j0, NLANE)] = idx_v[pl.ds(j0, NLANE)] // 8

    @pl.loop(0, BPW, step=NLANE)
    def _(j0):
        vals = idx_v[pl.ds(j0, NLANE)]
        for h in range(2):
            pltpu.sync_copy(mem3_hbm.at[gidx_v.at[pl.ds(j0 + 8 * h, 8)]],
                            grp_v)
            for k in range(8):
                slot = vals[8 * h + k]
                _start_row_dma(grp_v, k, slot % 8,
                               rows_s.at[sid], h, k, sem_r)

            for k in range(8):
                pltpu.make_async_copy(
                    grp_v.at[pl.ds(0, 1), pl.ds(0, 1)],
                    rows_s.at[pl.ds(sid, 1), pl.ds(h, 1), pl.ds(k, 1)],
                    sem_r,
                ).wait()

        pltpu.sync_copy(rows_s.at[sid], xrec_hbm.at[pl.ds((base + j0) // 8, 2)])


TB = 512
NTB = 196


def _trans_body(in_ref, out_ref):
    out_ref[...] = in_ref[...].T


def _transpose_to_rm(at, nrows):
    nblk = (nrows + TB - 1) // TB
    return pl.pallas_call(
        _trans_body,
        grid=(nblk,),
        in_specs=[pl.BlockSpec((D, TB), lambda i: (0, i))],
        out_specs=pl.BlockSpec((TB, D), lambda i: (i, 0)),
        out_shape=jax.ShapeDtypeStruct((nrows, D), jnp.float32),
    )(at)


def _trans_pad_body(in_ref, out_ref):
    t = in_ref[...].T
    z = jnp.zeros((TB, DP - D), jnp.float32)
    out_ref[...] = jnp.concatenate([t, z], axis=1)


def _transpose_pad(at, nrows):
    nblk = (nrows + TB - 1) // TB
    return pl.pallas_call(
        _trans_pad_body,
        grid=(nblk,),
        in_specs=[pl.BlockSpec((D, TB), lambda i: (0, i))],
        out_specs=pl.BlockSpec((TB, DP), lambda i: (i, 0)),
        out_shape=jax.ShapeDtypeStruct((nrows, DP), jnp.float32),
    )(at)


def _mg_body(wcnt_s, wsl_s, wsr_s, memrm_ref, x_hbm, dep_ref, out_ref, sem):
    b = pl.program_id(0)
    n = wcnt_s[(b // 7) * NLANE + (b % 7)]

    @pl.loop(0, n)
    def _(k):
        j = b * TBM + wsl_s[k]
        t = wsr_s[k]
        pltpu.make_async_copy(
            x_hbm.at[pl.ds(t, 1)], out_ref.at[pl.ds(j, 1)], sem).start()

    @pl.loop(0, n)
    def _(k):
        pltpu.make_async_copy(
            x_hbm.at[pl.ds(0, 1)], out_ref.at[pl.ds(0, 1)], sem).wait()


def _k_mg(wsl, wsr, wcnt, mem_rm, x_rm, dep):
    return pl.pallas_call(
        _mg_body,
        grid=(NBM,),
        in_specs=[
            pl.BlockSpec(memory_space=pltpu.SMEM),
            pl.BlockSpec((WREG,), lambda i: (i,), memory_space=pltpu.SMEM),
            pl.BlockSpec((WREG,), lambda i: (i,), memory_space=pltpu.SMEM),
            pl.BlockSpec(memory_space=pl.ANY),
            pl.BlockSpec(memory_space=pl.ANY),
            pl.BlockSpec(memory_space=pl.ANY),
        ],
        out_specs=pl.BlockSpec(memory_space=pl.ANY),
        out_shape=jax.ShapeDtypeStruct((MEMROWS, DP), jnp.float32),
        scratch_shapes=[pltpu.SemaphoreType.DMA],
        input_output_aliases={3: 0},
        compiler_params=pltpu.CompilerParams(has_side_effects=True),
    )(wcnt, wsl, wsr, mem_rm, x_rm, dep)


def _trout_body(in_ref, out_ref):
    out_ref[...] = in_ref[...][:, :D].T


def _k_trout(mem_merged):
    return pl.pallas_call(
        _trout_body,
        grid=(NTB,),
        in_specs=[pl.BlockSpec((TB, DP), lambda i: (i, 0))],
        out_specs=pl.BlockSpec((D, TB), lambda i: (0, i)),
        out_shape=jax.ShapeDtypeStruct((D, MEMROWS), jnp.float32),
    )(mem_merged)


def _k_mlp_body(x_ref, xrec_ref, w1_ref, b1_ref, w2_ref, b2_ref, wrec_ref,
                brec_ref, wout_ref, bout_ref, out_ref):
    act = jnp.maximum(
        jnp.dot(x_ref[...], w1_ref[...],
                preferred_element_type=jnp.float32) + b1_ref[...], 0.0)
    a2 = (jnp.dot(act, w2_ref[...],
                  preferred_element_type=jnp.float32) + b2_ref[...]
          + jnp.dot(xrec_ref[...], wrec_ref[...],
                    preferred_element_type=jnp.float32) + brec_ref[...])
    a2 = jnp.maximum(a2, 0.0)
    out_ref[...] = jnp.dot(a2, wout_ref[...],
                           preferred_element_type=jnp.float32) + bout_ref[...]


def _k_mlp(x, xrec, w1, b1, w2, b2, wrec, brec, wout, bout):
    full = lambda a: pl.BlockSpec(a.shape, lambda i: (0,) * a.ndim)
    return pl.pallas_call(
        _k_mlp_body,
        grid=(BATCH // BM,),
        in_specs=[
            pl.BlockSpec((BM, DP), lambda i: (i, 0)),
            pl.BlockSpec((BM, DP), lambda i: (i, 0)),
            full(w1), full(b1), full(w2), full(b2),
            full(wrec), full(brec), full(wout), full(bout),
        ],
        out_specs=pl.BlockSpec((BM, 10), lambda i: (i, 0)),
        out_shape=jax.ShapeDtypeStruct((BATCH, 10), jnp.float32),
    )(x, xrec, w1, b1, w2, b2, wrec, brec, wout, bout)


def kernel(x_sensory, mem_vals, W1, b1, W2, b2, Wrec, brec, Wout, bout):
    h = lax.stop_gradient(jax.nn.relu(x_sensory @ W1 + b1))
    mult = jnp.arange(1, h.shape[1] + 1, dtype=jnp.float32) * 2654435.0
    code = jnp.floor(h * 8.0) @ mult
    idx = jnp.mod(jnp.abs(code), float(MEMROWS))
    idx = jnp.clip(idx.astype(jnp.int32), 0, MEMROWS - 1)

    mem_rm = _transpose_pad(mem_vals.T, MEMROWS)
    x_rm = _transpose_pad(x_sensory.T, BATCH)

    wsl, wsr, wcnt = _k_tab(idx)
    xrec3 = _k_ga(mem_rm.reshape(MEMG, 8, DP), idx)

    w1_pad = jnp.pad(W1, ((0, DP - D), (0, 0)))
    wrec_pad = jnp.pad(Wrec, ((0, DP - D), (0, 0)))
    out = _k_mlp(x_rm, xrec3.reshape(BATCH, DP), w1_pad, b1.reshape(1, -1),
                 W2, b2.reshape(1, -1), wrec_pad, brec.reshape(1, -1), Wout,
                 bout.reshape(1, -1))

    mem_merged = _k_mg(wsl, wsr, wcnt, mem_rm, x_rm, xrec3)
    mem_newT = _k_trout(mem_merged)
    return out, mem_newT.T
